# baseline (speedup 1.0000x reference)
import functools

import jax
import jax.numpy as jnp
from jax import lax
from jax.experimental import pallas as pl
from jax.experimental.pallas import tpu as pltpu
from jax.experimental.pallas import tpu_sc as plsc

_N = 10000
_E = 320000
_D = 128
_H = 256
_NG = 20

_NC = 2
_NS = 16
_NPAD = 10240
_EPAD = 327680
_TILE_EDGES = _EPAD // (_NC * _NS)
_B = 128
_ROWS_PER_TILE = _NPAD // _NS
_HR = _NPAD // 128

_BN = 1024
_NEG = -1e30


def _sc_mesh():
    return plsc.VectorSubcoreMesh(core_axis_name="c", subcore_axis_name="s")


def _hist_call(dst_p):
    @functools.partial(
        pl.kernel,
        out_type=jax.ShapeDtypeStruct((_NC, _HR, 128), jnp.float32),
        mesh=_sc_mesh(),
        scratch_types=[
            pltpu.VMEM((_TILE_EDGES,), jnp.int32),
            pltpu.VMEM((_HR, 128), jnp.float32),
            pltpu.VMEM((_HR,), jnp.int32),
            pltpu.VMEM_SHARED((_HR, 128), jnp.float32),
        ],
        compiler_params=pltpu.CompilerParams(needs_layout_passes=False),
    )
    def hist_kernel(dst_hbm, out_hbm, dst_loc, hist, rowidx, deg_sh):
        c = lax.axis_index("c")
        s = lax.axis_index("s")
        wid = s * _NC + c

        zeros16 = jnp.zeros((16,), jnp.float32)

        def zrow(r, carry):
            for k in range(8):
                hist[r, pl.ds(k * 16, 16)] = zeros16
            return carry

        lax.fori_loop(0, _HR, zrow, 0)

        @pl.when(s == 0)
        def _():
            pltpu.sync_copy(hist, deg_sh)

        for r in range(_HR // 16):
            rowidx[pl.ds(r * 16, 16)] = lax.iota(jnp.int32, 16) + r * 16

        plsc.subcore_barrier()

        pltpu.sync_copy(dst_hbm.at[pl.ds(wid * _TILE_EDGES, _TILE_EDGES)], dst_loc)

        ones16 = jnp.ones((16,), jnp.float32)

        def ebody(i, carry):
            d = dst_loc[pl.ds(i * 16, 16)]
            plsc.addupdate_scatter(
                hist,
                [lax.shift_right_logical(d, 7), lax.bitwise_and(d, 127)],
                ones16,
            )
            return carry

        lax.fori_loop(0, _TILE_EDGES // 16, ebody, 0)

        pltpu.sync_copy(hist, deg_sh.at[rowidx], add=True)
        plsc.subcore_barrier()

        @pl.when(s == 0)
        def _():
            pltpu.sync_copy(deg_sh, out_hbm.at[c])

    return hist_kernel(dst_p)


_SB = 64
_NBUF = 4
_SUPB = 16
_SGRP = _SUPB // _NBUF
_T0B = 80
_T1B = 240
_S0 = _T0B // _SUPB
_S1 = _T1B // _SUPB


def _scatter_call(u, src_p, dst_p):
    src2 = src_p.reshape(_EPAD // _SB, _SB)
    dst2 = dst_p.reshape(_EPAD // _SB, _SB)

    @functools.partial(
        pl.kernel,
        out_type=jax.ShapeDtypeStruct((_NC, _NPAD, _D), jnp.float32),
        mesh=_sc_mesh(),
        scratch_types=[
            [pltpu.VMEM((_SUPB, _SB), jnp.int32)] * 2,
            [pltpu.VMEM((_SUPB, _SB), jnp.int32)] * 2,
            [pltpu.VMEM((_SB, _D), jnp.float32)] * _NBUF,
            pltpu.VMEM_SHARED((_NPAD, _D), jnp.float32),
            [pltpu.SemaphoreType.DMA] * _NBUF,
            [pltpu.SemaphoreType.DMA] * _NBUF,
            [pltpu.SemaphoreType.DMA] * 2,
        ],
    )
    def scatter_kernel(u_hbm, src_hbm, dst_hbm, out_hbm,
                       sidx, didx, bufs, acc_sh, gsems, ssems, isems):
        c = lax.axis_index("c")
        s = lax.axis_index("s")
        wid = s * _NC + c

        zeros16 = jnp.zeros((16,), jnp.float32)

        def zrow(r, carry):
            for k in range(_D // 16):
                bufs[0][r, pl.ds(k * 16, 16)] = zeros16
            return carry

        lax.fori_loop(0, _SB, zrow, 0)

        nz = _ROWS_PER_TILE // _SB
        for k in range(nz):
            pltpu.async_copy(
                bufs[0], acc_sh.at[pl.ds(s * _ROWS_PER_TILE + k * _SB, _SB)],
                gsems[k % _NBUF])
        for k in range(nz):
            pltpu.make_async_copy(
                bufs[0], acc_sh.at[pl.ds(s * _ROWS_PER_TILE + k * _SB, _SB)],
                gsems[k % _NBUF]).wait()

        ibase = jnp.where(c == 0, s * _T0B, 16 * _T0B + s * _T1B)
        nsup = jnp.where(c == 0, _S0, _S1)

        def idx_fetch(sp):
            st = sp % 2
            pltpu.async_copy(src_hbm.at[pl.ds(ibase + sp * _SUPB, _SUPB)], sidx[st], isems[st])
            pltpu.async_copy(dst_hbm.at[pl.ds(ibase + sp * _SUPB, _SUPB)], didx[st], isems[st])

        idx_fetch(0)
        idx_fetch(1)
        plsc.subcore_barrier()

        for sp in range(_S1):
            @pl.when(sp < nsup)
            def _(sp=sp):
                st = sp % 2
                pltpu.make_async_copy(src_hbm.at[pl.ds(ibase + sp * _SUPB, _SUPB)],
                                      sidx[st], isems[st]).wait()
                pltpu.make_async_copy(dst_hbm.at[pl.ds(ibase + sp * _SUPB, _SUPB)],
                                      didx[st], isems[st]).wait()

                def group(g, carry, st=st):
                    for b in range(_NBUF):
                        j = g * _NBUF + b

                        @pl.when(g > 0)
                        def _():
                            jp = j - _NBUF
                            pltpu.make_async_copy(
                                bufs[b], acc_sh.at[didx[st].at[jp]], ssems[b]).wait()

                        pltpu.async_copy(u_hbm.at[sidx[st].at[j]], bufs[b], gsems[b])
                    for b in range(_NBUF):
                        j = g * _NBUF + b
                        pltpu.make_async_copy(
                            u_hbm.at[sidx[st].at[j]], bufs[b], gsems[b]).wait()
                        pltpu.async_copy(bufs[b], acc_sh.at[didx[st].at[j]], ssems[b], add=True)
                    return carry

                lax.fori_loop(0, _SGRP, group, 0)
                for b in range(_NBUF):
                    j = (_SGRP - 1) * _NBUF + b
                    pltpu.make_async_copy(bufs[b], acc_sh.at[didx[st].at[j]], ssems[b]).wait()

                @pl.when(sp + 2 < nsup)
                def _():
                    idx_fetch(sp + 2)

        plsc.subcore_barrier()
        pltpu.sync_copy(
            acc_sh.at[pl.ds(s * _ROWS_PER_TILE, _ROWS_PER_TILE)],
            out_hbm.at[c, pl.ds(s * _ROWS_PER_TILE, _ROWS_PER_TILE)],
        )

    return scatter_kernel(u, src2, dst2)


def _u_call(deg_col, x_pad):
    def body(deg_ref, x_ref, u_ref):
        u_ref[...] = lax.rsqrt(deg_ref[...]) * x_ref[...]

    return pl.pallas_call(
        body,
        grid=(_NPAD // _BN,),
        in_specs=[
            pl.BlockSpec((_BN, 1), lambda i: (i, 0)),
            pl.BlockSpec((_BN, _D), lambda i: (i, 0)),
        ],
        out_specs=pl.BlockSpec((_BN, _D), lambda i: (i, 0)),
        out_shape=jax.ShapeDtypeStruct((_NPAD, _D), jnp.float32),
    )(deg_col, x_pad)


def _lrelu(v):
    return jnp.where(v > 0, v, 0.2 * v)


def _dense_body(sp_ref, x_ref, deg_ref, z_ref, t_ref,
                wg_ref, bg_ref, w1a_ref, w1b_ref, b1_ref,
                w2_ref, b2_ref, w3_ref, b3_ref,
                wd1_ref, bd1_ref, wd3_ref, bd3_ref,
                wz_ref, bz_ref,
                wq1a_ref, wq1az_ref, bq1a_ref, wq1b_ref, bq1b_ref, wq1c_ref, bq1c_ref,
                wq0a_ref, wq0az_ref, bq0a_ref, wq0b_ref, bq0b_ref, wq0c_ref, bq0c_ref,
                gt_ref, gz_ref, q_ref):
    f32 = jnp.float32
    x = x_ref[...]
    dinv = lax.rsqrt(deg_ref[...])
    s_sum = sp_ref[0] + sp_ref[1]
    agg = dinv * s_sum + (dinv * dinv) * x
    emb0 = jnp.dot(agg, wg_ref[...], preferred_element_type=f32) + bg_ref[...]
    h1 = _lrelu(jnp.dot(emb0, w1a_ref[...], preferred_element_type=f32)
                + jnp.dot(x, w1b_ref[...], preferred_element_type=f32) + b1_ref[...])
    h2 = _lrelu(jnp.dot(h1, w2_ref[...], preferred_element_type=f32) + b2_ref[...])
    emb = jnp.dot(h2, w3_ref[...], preferred_element_type=f32) + b3_ref[...]

    d1 = _lrelu(jnp.dot(emb, wd1_ref[...], preferred_element_type=f32) + bd1_ref[...])
    gt_logit = jnp.dot(d1, wd3_ref[...], preferred_element_type=f32)[:, 0:1] + bd3_ref[...]
    gt_ref[...] = 1.0 / (1.0 + jnp.exp(-gt_logit))

    logits = jnp.dot(emb, wz_ref[...], preferred_element_type=f32) + bz_ref[...]
    m = jnp.max(logits, axis=1, keepdims=True)
    e = jnp.exp(logits - m)
    probs = e / jnp.sum(e, axis=1, keepdims=True)
    z = z_ref[...]
    zn = z * float(_NG)
    up = jnp.ceil(zn)
    inter = 1.0 - (up - zn)
    lo = up - 1.0
    lo = lo + (lo < 0).astype(f32)
    cols = lax.broadcasted_iota(jnp.int32, (_BN, 128), 1)
    lo_i = lo.astype(jnp.int32)
    up_i = up.astype(jnp.int32)
    l_out = jnp.sum(jnp.where(cols == lo_i, probs, 0.0), axis=1, keepdims=True)
    u_out = jnp.sum(jnp.where(cols == up_i, probs, 0.0), axis=1, keepdims=True)
    gz_ref[...] = l_out + (u_out - l_out) * inter

    ha = _lrelu(jnp.dot(emb, wq1a_ref[...], preferred_element_type=f32)
                + z * wq1az_ref[...] + bq1a_ref[...])
    hb = _lrelu(jnp.dot(ha, wq1b_ref[...], preferred_element_type=f32) + bq1b_ref[...])
    q1 = jnp.dot(hb, wq1c_ref[...], preferred_element_type=f32)[:, 0:1] + bq1c_ref[...]
    ha0 = _lrelu(jnp.dot(emb, wq0a_ref[...], preferred_element_type=f32)
                 + z * wq0az_ref[...] + bq0a_ref[...])
    hb0 = _lrelu(jnp.dot(ha0, wq0b_ref[...], preferred_element_type=f32) + bq0b_ref[...])
    q0 = jnp.dot(hb0, wq0c_ref[...], preferred_element_type=f32)[:, 0:1] + bq0c_ref[...]
    tt = t_ref[...]
    q_ref[...] = tt * q1 + (1.0 - tt) * q0


def _dense_call(sp, x_pad, deg_col, z_col, t_col, weights):
    def full(shape):
        return pl.BlockSpec(shape, lambda i: tuple(0 for _ in shape))

    col = pl.BlockSpec((_BN, 1), lambda i: (i, 0))
    in_specs = [
        pl.BlockSpec((_NC, _BN, _D), lambda i: (0, i, 0)),
        pl.BlockSpec((_BN, _D), lambda i: (i, 0)),
        col, col, col,
    ] + [full(w.shape) for w in weights]
    out_shape = jax.ShapeDtypeStruct((_NPAD, 1), jnp.float32)
    return pl.pallas_call(
        _dense_body,
        grid=(_NPAD // _BN,),
        in_specs=in_specs,
        out_specs=[col, col, col],
        out_shape=[out_shape, out_shape, out_shape],
    )(sp, x_pad, deg_col, z_col, t_col, *weights)


def kernel(x, t, z, edge_index, W_gcn, b_gcn, W1, b1, W2, b2, W3, b3,
           Wd1, bd1, Wd3, bd3, Wz, bz, Wq1a, bq1a, Wq1b, bq1b, Wq1c, bq1c,
           Wq0a, bq0a, Wq0b, bq0b, Wq0c, bq0c):
    f32 = jnp.float32
    src = edge_index[0]
    dst = edge_index[1]
    idx_pad = jnp.full((_EPAD - _E,), _N, jnp.int32)
    src_p = jnp.concatenate([src, idx_pad])
    dst_p = jnp.concatenate([dst, idx_pad])
    x_pad = jnp.concatenate([x, jnp.zeros((_NPAD - _N, _D), f32)], axis=0)

    degp = _hist_call(dst_p)
    deg_col = (degp[0] + degp[1]).reshape(_NPAD, 1) + 1.0
    u = _u_call(deg_col, x_pad)
    sp = _scatter_call(u, src_p, dst_p)

    def colpad(v):
        return jnp.concatenate([v, jnp.zeros((_NPAD - _N,), f32)])[:, None]

    z_col = colpad(z)
    t_col = colpad(t)

    wd3T = jnp.zeros((_H, 128), f32).at[:, 0].set(Wd3[0])
    wzp = jnp.zeros((128, 128), f32).at[:, : _NG + 1].set(Wz)
    bzp = jnp.full((1, 128), _NEG, f32).at[0, : _NG + 1].set(bz)
    wq1aT = Wq1a.T
    wq0aT = Wq0a.T
    wq1cT = jnp.zeros((_H, 128), f32).at[:, 0].set(Wq1c[0])
    wq0cT = jnp.zeros((_H, 128), f32).at[:, 0].set(Wq0c[0])

    weights = [
        W_gcn.T, b_gcn[None, :],
        W1[:, :_H].T, W1[:, _H:].T, b1[None, :],
        W2.T, b2[None, :], W3.T, b3[None, :],
        Wd1.T, bd1[None, :], wd3T, bd3[None, :],
        wzp, bzp,
        wq1aT[:_D], wq1aT[_D:], bq1a[None, :], Wq1b.T, bq1b[None, :], wq1cT, bq1c[None, :],
        wq0aT[:_D], wq0aT[_D:], bq0a[None, :], Wq0b.T, bq0b[None, :], wq0cT, bq0c[None, :],
    ]

    gt, gz, q = _dense_call(sp, x_pad, deg_col, z_col, t_col, weights)
    return gt[:_N], gz[:_N], q[:_N]


# speedup vs baseline: 1.1630x; 1.1630x over previous
import functools

import jax
import jax.numpy as jnp
from jax import lax
from jax.experimental import pallas as pl
from jax.experimental.pallas import tpu as pltpu
from jax.experimental.pallas import tpu_sc as plsc

_N = 10000
_E = 320000
_D = 128
_H = 256
_NG = 20

_NC = 2
_NS = 16
_NPAD = 10240
_EPAD = 327680
_TILE_EDGES = _EPAD // (_NC * _NS)
_B = 128
_ROWS_PER_TILE = _NPAD // _NS
_HR = _NPAD // 128

_BN = 1024
_NEG = -1e30


def _sc_mesh():
    return plsc.VectorSubcoreMesh(core_axis_name="c", subcore_axis_name="s")


def _hist_call(dst_p):
    @functools.partial(
        pl.kernel,
        out_type=jax.ShapeDtypeStruct((_NC, _HR, 128), jnp.float32),
        mesh=_sc_mesh(),
        scratch_types=[
            pltpu.VMEM((_TILE_EDGES,), jnp.int32),
            pltpu.VMEM((_HR, 128), jnp.float32),
            pltpu.VMEM((_HR,), jnp.int32),
            pltpu.VMEM_SHARED((_HR, 128), jnp.float32),
        ],
        compiler_params=pltpu.CompilerParams(needs_layout_passes=False),
    )
    def hist_kernel(dst_hbm, out_hbm, dst_loc, hist, rowidx, deg_sh):
        c = lax.axis_index("c")
        s = lax.axis_index("s")
        wid = s * _NC + c

        zeros16 = jnp.zeros((16,), jnp.float32)

        def zrow(r, carry):
            for k in range(8):
                hist[r, pl.ds(k * 16, 16)] = zeros16
            return carry

        lax.fori_loop(0, _HR, zrow, 0)

        @pl.when(s == 0)
        def _():
            pltpu.sync_copy(hist, deg_sh)

        for r in range(_HR // 16):
            rowidx[pl.ds(r * 16, 16)] = lax.iota(jnp.int32, 16) + r * 16

        plsc.subcore_barrier()

        pltpu.sync_copy(dst_hbm.at[pl.ds(wid * _TILE_EDGES, _TILE_EDGES)], dst_loc)

        ones16 = jnp.ones((16,), jnp.float32)

        def ebody(i, carry):
            d = dst_loc[pl.ds(i * 16, 16)]
            plsc.addupdate_scatter(
                hist,
                [lax.shift_right_logical(d, 7), lax.bitwise_and(d, 127)],
                ones16,
            )
            return carry

        lax.fori_loop(0, _TILE_EDGES // 16, ebody, 0)

        pltpu.sync_copy(hist, deg_sh.at[rowidx], add=True)
        plsc.subcore_barrier()

        @pl.when(s == 0)
        def _():
            pltpu.sync_copy(deg_sh, out_hbm.at[c])

    return hist_kernel(dst_p)


_SB = 64
_NBUF = 4
_SUPB = 16
_SGRP = _SUPB // _NBUF
_T0B = 240
_T1B = 80
_S0 = _T0B // _SUPB
_S1 = _T1B // _SUPB


def _scatter_call(u, src_p, dst_p):
    src2 = src_p.reshape(_EPAD // _SB, _SB)
    dst2 = dst_p.reshape(_EPAD // _SB, _SB)

    @functools.partial(
        pl.kernel,
        out_type=jax.ShapeDtypeStruct((_NC, _NPAD, _D), jnp.float32),
        mesh=_sc_mesh(),
        scratch_types=[
            [pltpu.VMEM((_SUPB, _SB), jnp.int32)] * 2,
            [pltpu.VMEM((_SUPB, _SB), jnp.int32)] * 2,
            [pltpu.VMEM((_SB, _D), jnp.float32)] * _NBUF,
            pltpu.VMEM_SHARED((_NPAD, _D), jnp.float32),
            [pltpu.SemaphoreType.DMA] * _NBUF,
            [pltpu.SemaphoreType.DMA] * _NBUF,
            [pltpu.SemaphoreType.DMA] * 2,
        ],
    )
    def scatter_kernel(u_hbm, src_hbm, dst_hbm, out_hbm,
                       sidx, didx, bufs, acc_sh, gsems, ssems, isems):
        c = lax.axis_index("c")
        s = lax.axis_index("s")
        wid = s * _NC + c

        zeros16 = jnp.zeros((16,), jnp.float32)

        def zrow(r, carry):
            for k in range(_D // 16):
                bufs[0][r, pl.ds(k * 16, 16)] = zeros16
            return carry

        lax.fori_loop(0, _SB, zrow, 0)

        nz = _ROWS_PER_TILE // _SB
        for k in range(nz):
            pltpu.async_copy(
                bufs[0], acc_sh.at[pl.ds(s * _ROWS_PER_TILE + k * _SB, _SB)],
                gsems[k % _NBUF])
        for k in range(nz):
            pltpu.make_async_copy(
                bufs[0], acc_sh.at[pl.ds(s * _ROWS_PER_TILE + k * _SB, _SB)],
                gsems[k % _NBUF]).wait()

        ibase = jnp.where(c == 0, s * _T0B, 16 * _T0B + s * _T1B)
        nsup = jnp.where(c == 0, _S0, _S1)

        def idx_fetch(sp):
            st = sp % 2
            pltpu.async_copy(src_hbm.at[pl.ds(ibase + sp * _SUPB, _SUPB)], sidx[st], isems[st])
            pltpu.async_copy(dst_hbm.at[pl.ds(ibase + sp * _SUPB, _SUPB)], didx[st], isems[st])

        idx_fetch(0)
        idx_fetch(1)
        plsc.subcore_barrier()

        for sp in range(_S1):
            @pl.when(sp < nsup)
            def _(sp=sp):
                st = sp % 2
                pltpu.make_async_copy(src_hbm.at[pl.ds(ibase + sp * _SUPB, _SUPB)],
                                      sidx[st], isems[st]).wait()
                pltpu.make_async_copy(dst_hbm.at[pl.ds(ibase + sp * _SUPB, _SUPB)],
                                      didx[st], isems[st]).wait()

                def group(g, carry, st=st):
                    for b in range(_NBUF):
                        j = g * _NBUF + b

                        @pl.when(g > 0)
                        def _():
                            jp = j - _NBUF
                            pltpu.make_async_copy(
                                bufs[b], acc_sh.at[didx[st].at[jp]], ssems[b]).wait()

                        pltpu.async_copy(u_hbm.at[sidx[st].at[j]], bufs[b], gsems[b])
                    for b in range(_NBUF):
                        j = g * _NBUF + b
                        pltpu.make_async_copy(
                            u_hbm.at[sidx[st].at[j]], bufs[b], gsems[b]).wait()
                        pltpu.async_copy(bufs[b], acc_sh.at[didx[st].at[j]], ssems[b], add=True)
                    return carry

                lax.fori_loop(0, _SGRP, group, 0)
                for b in range(_NBUF):
                    j = (_SGRP - 1) * _NBUF + b
                    pltpu.make_async_copy(bufs[b], acc_sh.at[didx[st].at[j]], ssems[b]).wait()

                @pl.when(sp + 2 < nsup)
                def _():
                    idx_fetch(sp + 2)

        plsc.subcore_barrier()
        pltpu.sync_copy(
            acc_sh.at[pl.ds(s * _ROWS_PER_TILE, _ROWS_PER_TILE)],
            out_hbm.at[c, pl.ds(s * _ROWS_PER_TILE, _ROWS_PER_TILE)],
        )

    return scatter_kernel(u, src2, dst2)


def _u_call(deg_col, x_pad):
    def body(deg_ref, x_ref, u_ref):
        u_ref[...] = lax.rsqrt(deg_ref[...]) * x_ref[...]

    return pl.pallas_call(
        body,
        grid=(_NPAD // _BN,),
        in_specs=[
            pl.BlockSpec((_BN, 1), lambda i: (i, 0)),
            pl.BlockSpec((_BN, _D), lambda i: (i, 0)),
        ],
        out_specs=pl.BlockSpec((_BN, _D), lambda i: (i, 0)),
        out_shape=jax.ShapeDtypeStruct((_NPAD, _D), jnp.float32),
    )(deg_col, x_pad)


def _lrelu(v):
    return jnp.where(v > 0, v, 0.2 * v)


def _dense_body(sp_ref, x_ref, deg_ref, z_ref, t_ref,
                wg_ref, bg_ref, w1a_ref, w1b_ref, b1_ref,
                w2_ref, b2_ref, w3_ref, b3_ref,
                wd1_ref, bd1_ref, wd3_ref, bd3_ref,
                wz_ref, bz_ref,
                wq1a_ref, wq1az_ref, bq1a_ref, wq1b_ref, bq1b_ref, wq1c_ref, bq1c_ref,
                wq0a_ref, wq0az_ref, bq0a_ref, wq0b_ref, bq0b_ref, wq0c_ref, bq0c_ref,
                gt_ref, gz_ref, q_ref):
    f32 = jnp.float32
    x = x_ref[...]
    dinv = lax.rsqrt(deg_ref[...])
    s_sum = sp_ref[0] + sp_ref[1]
    agg = dinv * s_sum + (dinv * dinv) * x
    emb0 = jnp.dot(agg, wg_ref[...], preferred_element_type=f32) + bg_ref[...]
    h1 = _lrelu(jnp.dot(emb0, w1a_ref[...], preferred_element_type=f32)
                + jnp.dot(x, w1b_ref[...], preferred_element_type=f32) + b1_ref[...])
    h2 = _lrelu(jnp.dot(h1, w2_ref[...], preferred_element_type=f32) + b2_ref[...])
    emb = jnp.dot(h2, w3_ref[...], preferred_element_type=f32) + b3_ref[...]

    d1 = _lrelu(jnp.dot(emb, wd1_ref[...], preferred_element_type=f32) + bd1_ref[...])
    gt_logit = jnp.dot(d1, wd3_ref[...], preferred_element_type=f32)[:, 0:1] + bd3_ref[...]
    gt_ref[...] = 1.0 / (1.0 + jnp.exp(-gt_logit))

    logits = jnp.dot(emb, wz_ref[...], preferred_element_type=f32) + bz_ref[...]
    m = jnp.max(logits, axis=1, keepdims=True)
    e = jnp.exp(logits - m)
    probs = e / jnp.sum(e, axis=1, keepdims=True)
    z = z_ref[...]
    zn = z * float(_NG)
    up = jnp.ceil(zn)
    inter = 1.0 - (up - zn)
    lo = up - 1.0
    lo = lo + (lo < 0).astype(f32)
    cols = lax.broadcasted_iota(jnp.int32, (_BN, 128), 1)
    lo_i = lo.astype(jnp.int32)
    up_i = up.astype(jnp.int32)
    l_out = jnp.sum(jnp.where(cols == lo_i, probs, 0.0), axis=1, keepdims=True)
    u_out = jnp.sum(jnp.where(cols == up_i, probs, 0.0), axis=1, keepdims=True)
    gz_ref[...] = l_out + (u_out - l_out) * inter

    ha = _lrelu(jnp.dot(emb, wq1a_ref[...], preferred_element_type=f32)
                + z * wq1az_ref[...] + bq1a_ref[...])
    hb = _lrelu(jnp.dot(ha, wq1b_ref[...], preferred_element_type=f32) + bq1b_ref[...])
    q1 = jnp.dot(hb, wq1c_ref[...], preferred_element_type=f32)[:, 0:1] + bq1c_ref[...]
    ha0 = _lrelu(jnp.dot(emb, wq0a_ref[...], preferred_element_type=f32)
                 + z * wq0az_ref[...] + bq0a_ref[...])
    hb0 = _lrelu(jnp.dot(ha0, wq0b_ref[...], preferred_element_type=f32) + bq0b_ref[...])
    q0 = jnp.dot(hb0, wq0c_ref[...], preferred_element_type=f32)[:, 0:1] + bq0c_ref[...]
    tt = t_ref[...]
    q_ref[...] = tt * q1 + (1.0 - tt) * q0


def _dense_call(sp, x_pad, deg_col, z_col, t_col, weights):
    def full(shape):
        return pl.BlockSpec(shape, lambda i: tuple(0 for _ in shape))

    col = pl.BlockSpec((_BN, 1), lambda i: (i, 0))
    in_specs = [
        pl.BlockSpec((_NC, _BN, _D), lambda i: (0, i, 0)),
        pl.BlockSpec((_BN, _D), lambda i: (i, 0)),
        col, col, col,
    ] + [full(w.shape) for w in weights]
    out_shape = jax.ShapeDtypeStruct((_NPAD, 1), jnp.float32)
    return pl.pallas_call(
        _dense_body,
        grid=(_NPAD // _BN,),
        in_specs=in_specs,
        out_specs=[col, col, col],
        out_shape=[out_shape, out_shape, out_shape],
    )(sp, x_pad, deg_col, z_col, t_col, *weights)


def kernel(x, t, z, edge_index, W_gcn, b_gcn, W1, b1, W2, b2, W3, b3,
           Wd1, bd1, Wd3, bd3, Wz, bz, Wq1a, bq1a, Wq1b, bq1b, Wq1c, bq1c,
           Wq0a, bq0a, Wq0b, bq0b, Wq0c, bq0c):
    f32 = jnp.float32
    src = edge_index[0]
    dst = edge_index[1]
    idx_pad = jnp.full((_EPAD - _E,), _N, jnp.int32)
    src_p = jnp.concatenate([src, idx_pad])
    dst_p = jnp.concatenate([dst, idx_pad])
    x_pad = jnp.concatenate([x, jnp.zeros((_NPAD - _N, _D), f32)], axis=0)

    degp = _hist_call(dst_p)
    deg_col = (degp[0] + degp[1]).reshape(_NPAD, 1) + 1.0
    u = _u_call(deg_col, x_pad)
    sp = _scatter_call(u, src_p, dst_p)

    def colpad(v):
        return jnp.concatenate([v, jnp.zeros((_NPAD - _N,), f32)])[:, None]

    z_col = colpad(z)
    t_col = colpad(t)

    wd3T = jnp.zeros((_H, 128), f32).at[:, 0].set(Wd3[0])
    wzp = jnp.zeros((128, 128), f32).at[:, : _NG + 1].set(Wz)
    bzp = jnp.full((1, 128), _NEG, f32).at[0, : _NG + 1].set(bz)
    wq1aT = Wq1a.T
    wq0aT = Wq0a.T
    wq1cT = jnp.zeros((_H, 128), f32).at[:, 0].set(Wq1c[0])
    wq0cT = jnp.zeros((_H, 128), f32).at[:, 0].set(Wq0c[0])

    weights = [
        W_gcn.T, b_gcn[None, :],
        W1[:, :_H].T, W1[:, _H:].T, b1[None, :],
        W2.T, b2[None, :], W3.T, b3[None, :],
        Wd1.T, bd1[None, :], wd3T, bd3[None, :],
        wzp, bzp,
        wq1aT[:_D], wq1aT[_D:], bq1a[None, :], Wq1b.T, bq1b[None, :], wq1cT, bq1c[None, :],
        wq0aT[:_D], wq0aT[_D:], bq0a[None, :], Wq0b.T, bq0b[None, :], wq0cT, bq0c[None, :],
    ]

    gt, gz, q = _dense_call(sp, x_pad, deg_col, z_col, t_col, weights)
    return gt[:_N], gz[:_N], q[:_N]
